# baseline (speedup 1.0000x reference)
import functools

import jax
import jax.numpy as jnp
from jax import lax
from jax.experimental import pallas as pl
from jax.experimental.pallas import tpu as pltpu
from jax.experimental.pallas import tpu_sc as plsc

_N = 10000
_E = 320000
_D = 128

_NW = 32
_ROWS = 313
_NPAD = _NW * _ROWS
_BATCH = 128
_EPT = _E // _NW
_CAP = 10112
_BSTRIDE = _BATCH + 16

_mesh = plsc.VectorSubcoreMesh(core_axis_name="c", subcore_axis_name="s")


def _wid():
    return lax.axis_index("s") * 2 + lax.axis_index("c")


def _splat(x):
    return jnp.full((16,), 1, jnp.int32) * x


@functools.partial(
    pl.kernel,
    out_type=[
        jax.ShapeDtypeStruct((_NW * _NW * _CAP,), jnp.int32),
        jax.ShapeDtypeStruct((_NW * _NW * _CAP,), jnp.int32),
        jax.ShapeDtypeStruct((_NW * _NW * 16,), jnp.int32),
    ],
    mesh=_mesh,
    scratch_types=[
        pltpu.VMEM((_EPT + 16,), jnp.int32),
        pltpu.VMEM((_EPT + 16,), jnp.int32),
        pltpu.VMEM((_NW * _BSTRIDE,), jnp.int32),
        pltpu.VMEM((_NW * _BSTRIDE,), jnp.int32),
        pltpu.VMEM((_NW * 16,), jnp.int32),
        pltpu.VMEM((_NW * 16,), jnp.int32),
        pltpu.VMEM((16,), jnp.int32),
    ],
)
def _bucketize(edges_hbm, bsrc_hbm, bdst_hbm, cnt_hbm, sstage, dstage, sbuf,
               dbuf, fill, done, cbuf):
    t = _wid()
    zero16 = jnp.zeros((16,), jnp.int32)

    def zinit(b, _):
        fill[pl.ds(pl.multiple_of(b * 16, 16), 16)] = zero16
        done[pl.ds(pl.multiple_of(b * 16, 16), 16)] = zero16
        return 0

    lax.fori_loop(0, _NW, zinit, 0)

    eo = pl.multiple_of(t * _EPT, 16)
    pltpu.sync_copy(edges_hbm.at[pl.ds(eo, _EPT)], sstage.at[pl.ds(0, _EPT)])
    pltpu.sync_copy(edges_hbm.at[pl.ds(_E + eo, _EPT)],
                    dstage.at[pl.ds(0, _EPT)])

    def edge(e, _):
        s = sstage[pl.ds(e, 16)][0]
        d = dstage[pl.ds(e, 16)][0]
        b = d // _ROWS
        loc = d - b * _ROWS
        c = fill[pl.ds(b * 16, 16)][0]
        sbuf[pl.ds(b * _BSTRIDE + c, 16)] = _splat(s)
        dbuf[pl.ds(b * _BSTRIDE + c, 16)] = _splat(loc)

        def flush(_):
            w0 = done[pl.ds(b * 16, 16)][0]
            o = pl.multiple_of((b * _NW + t) * _CAP + w0, 8)
            bo = pl.multiple_of(b * _BSTRIDE, 8)
            pltpu.sync_copy(sbuf.at[pl.ds(bo, _BATCH)],
                            bsrc_hbm.at[pl.ds(o, _BATCH)])
            pltpu.sync_copy(dbuf.at[pl.ds(bo, _BATCH)],
                            bdst_hbm.at[pl.ds(o, _BATCH)])
            done[pl.ds(b * 16, 16)] = _splat(w0 + _BATCH)
            fill[pl.ds(b * 16, 16)] = zero16
            return 0

        def keep(_):
            fill[pl.ds(b * 16, 16)] = _splat(c + 1)
            return 0

        lax.cond(c + 1 >= _BATCH, flush, keep, 0)
        return 0

    lax.fori_loop(0, _EPT, edge, 0)

    pad_d = jnp.full((16,), _ROWS, jnp.int32)

    def fin(b, _):
        c = fill[pl.ds(b * 16, 16)][0]
        for j in range(8):
            @pl.when(c + 16 * j < _BATCH)
            def _():
                sbuf[pl.ds(b * _BSTRIDE + c + 16 * j, 16)] = zero16
                dbuf[pl.ds(b * _BSTRIDE + c + 16 * j, 16)] = pad_d
        w0 = done[pl.ds(b * 16, 16)][0]
        o = pl.multiple_of((b * _NW + t) * _CAP + w0, 8)
        bo = pl.multiple_of(b * _BSTRIDE, 8)
        pltpu.sync_copy(sbuf.at[pl.ds(bo, _BATCH)],
                        bsrc_hbm.at[pl.ds(o, _BATCH)])
        pltpu.sync_copy(dbuf.at[pl.ds(bo, _BATCH)],
                        bdst_hbm.at[pl.ds(o, _BATCH)])
        cbuf[pl.ds(0, 16)] = _splat(w0 + c)
        co = pl.multiple_of((b * _NW + t) * 16, 16)
        pltpu.sync_copy(cbuf, cnt_hbm.at[pl.ds(co, 16)])
        return 0

    lax.fori_loop(0, _NW, fin, 0)


@functools.partial(
    pl.kernel,
    out_type=[
        jax.ShapeDtypeStruct((_NPAD * _D,), jnp.float32),
        jax.ShapeDtypeStruct((_NPAD * 16,), jnp.float32),
    ],
    mesh=_mesh,
    scratch_types=[
        pltpu.VMEM(((_ROWS + 1) * _D,), jnp.float32),
        pltpu.VMEM(((_ROWS + 1) * 16,), jnp.float32),
        pltpu.VMEM((_BATCH,), jnp.int32),
        pltpu.VMEM((_BATCH + 16,), jnp.int32),
        pltpu.VMEM((_BATCH, _D), jnp.float32),
        pltpu.VMEM((_NW * 16,), jnp.int32),
        pltpu.SemaphoreType.DMA,
    ],
)
def _segsum(m_hbm, bsrc_hbm, bdst_hbm, cnt_hbm, out_hbm, deg_hbm, acc, dacc,
            sidx, didx, rows, cbuf, sem):
    w = _wid()
    zero = jnp.zeros((16,), jnp.float32)
    ones = jnp.ones((16,), jnp.float32)

    def zbody(r, _):
        for j in range(_D // 16):
            acc[pl.ds(r * _D + 16 * j, 16)] = zero
        dacc[pl.ds(r * 16, 16)] = zero
        return 0

    lax.fori_loop(0, _ROWS + 1, zbody, 0)

    pltpu.sync_copy(
        cnt_hbm.at[pl.ds(pl.multiple_of(w * _NW * 16, 16), _NW * 16)], cbuf)

    def seg(tt, _):
        cnt = cbuf[pl.ds(tt * 16, 16)][0]
        nb = (cnt + (_BATCH - 1)) // _BATCH
        segbase = (w * _NW + tt) * _CAP

        def batch(b, _):
            o = pl.multiple_of(segbase + b * _BATCH, 8)
            pltpu.sync_copy(bsrc_hbm.at[pl.ds(o, _BATCH)], sidx)
            pltpu.sync_copy(bdst_hbm.at[pl.ds(o, _BATCH)],
                            didx.at[pl.ds(0, _BATCH)])
            pltpu.async_copy(m_hbm.at[sidx], rows, sem).wait()

            def edge(e, _):
                d = didx[pl.ds(e, 16)][0]
                for j in range(_D // 16):
                    plsc.addupdate(acc.at[pl.ds(d * _D + 16 * j, 16)],
                                   rows[e, pl.ds(16 * j, 16)])
                plsc.addupdate(dacc.at[pl.ds(d * 16, 16)], ones)
                return 0

            lax.fori_loop(0, _BATCH, edge, 0)
            return 0

        lax.fori_loop(0, nb, batch, 0)
        return 0

    lax.fori_loop(0, _NW, seg, 0)

    pltpu.sync_copy(acc.at[pl.ds(0, _ROWS * _D)],
                    out_hbm.at[pl.ds(pl.multiple_of(w * _ROWS * _D, 64),
                                     _ROWS * _D)])
    pltpu.sync_copy(dacc.at[pl.ds(0, _ROWS * 16)],
                    deg_hbm.at[pl.ds(pl.multiple_of(w * _ROWS * 16, 16),
                                     _ROWS * 16)])


_R = 1000


def _gelu(x):
    return 0.5 * x * (1.0 + lax.erf(x * 0.7071067811865476))


def _mm(a, b):
    return jnp.dot(a, b, preferred_element_type=jnp.float32)


def _head_body(f_ref, Win_ref, bin_ref, W1_ref, b1_ref, h_ref, m_ref):
    h = _gelu(_mm(f_ref[...], Win_ref[...]) + bin_ref[...])
    h_ref[...] = h
    m_ref[...] = _gelu(_mm(h, W1_ref[...]) + b1_ref[...])


def _mid_body(s_ref, dg_ref, h_ref, W2_ref, b2_ref, W3_ref, b3_ref, Wl_ref,
              bl_ref, W1n_ref, b1n_ref, hn_ref, mn_ref):
    h = h_ref[...]
    inv = 1.0 / jnp.maximum(dg_ref[...][:, :1], 1.0)
    agg = s_ref[...] * inv
    t = _gelu(_mm(agg, W2_ref[...]) + b2_ref[...] + _mm(h, W3_ref[...]) +
              b3_ref[...])
    hn = _mm(t, Wl_ref[...]) + bl_ref[...] + h
    hn_ref[...] = hn
    mn_ref[...] = _gelu(_mm(hn, W1n_ref[...]) + b1n_ref[...])


def _tail_body(s_ref, dg_ref, h_ref, W2_ref, b2_ref, W3_ref, b3_ref, Wl_ref,
               bl_ref, Wo_ref, bo_ref, o_ref):
    h = h_ref[...]
    inv = 1.0 / jnp.maximum(dg_ref[...][:, :1], 1.0)
    agg = s_ref[...] * inv
    t = _gelu(_mm(agg, W2_ref[...]) + b2_ref[...] + _mm(h, W3_ref[...]) +
              b3_ref[...])
    hn = _mm(t, Wl_ref[...]) + bl_ref[...] + h
    o_ref[...] = _mm(hn, Wo_ref[...]) + bo_ref[...]


_rows_spec = pl.BlockSpec((_R, _D), lambda i: (i, 0))
_deg_spec = pl.BlockSpec((_R, 16), lambda i: (i, 0))
_w_spec = pl.BlockSpec((_D, _D), lambda i: (0, 0))
_b_spec = pl.BlockSpec((_D,), lambda i: (0,))
_row_out = jax.ShapeDtypeStruct((_N, _D), jnp.float32)

_head = pl.pallas_call(
    _head_body,
    grid=(_N // _R,),
    in_specs=[_rows_spec, _w_spec, _b_spec, _w_spec, _b_spec],
    out_specs=[_rows_spec, _rows_spec],
    out_shape=[_row_out, _row_out],
)

_mid = pl.pallas_call(
    _mid_body,
    grid=(_N // _R,),
    in_specs=[_rows_spec, _deg_spec, _rows_spec] + [_w_spec, _b_spec] * 4,
    out_specs=[_rows_spec, _rows_spec],
    out_shape=[_row_out, _row_out],
)

_tail = pl.pallas_call(
    _tail_body,
    grid=(_N // _R,),
    in_specs=[_rows_spec, _deg_spec, _rows_spec] + [_w_spec, _b_spec] * 4,
    out_specs=_rows_spec,
    out_shape=_row_out,
)


def kernel(feats, edge_index, W_in, b_in,
           W1_0, b1_0, W2_0, b2_0, W3_0, b3_0, Wl_0, bl_0,
           W1_1, b1_1, W2_1, b2_1, W3_1, b3_1, Wl_1, bl_1,
           W_out, b_out):
    bsrc, bdst, cnts = _bucketize(edge_index.reshape(-1))
    h, m0 = _head(feats, W_in, b_in, W1_0, b1_0)
    s0, dg = _segsum(m0, bsrc, bdst, cnts)
    dg = dg.reshape(_NPAD, 16)[:_N]
    h1, m1 = _mid(s0.reshape(_NPAD, _D)[:_N], dg, h, W2_0, b2_0, W3_0, b3_0,
                  Wl_0, bl_0, W1_1, b1_1)
    s1, _ = _segsum(m1, bsrc, bdst, cnts)
    out = _tail(s1.reshape(_NPAD, _D)[:_N], dg, h1, W2_1, b2_1, W3_1, b3_1,
                Wl_1, bl_1, W_out, b_out)
    return out


# speedup vs baseline: 1.0025x; 1.0025x over previous
import functools

import jax
import jax.numpy as jnp
from jax import lax
from jax.experimental import pallas as pl
from jax.experimental.pallas import tpu as pltpu
from jax.experimental.pallas import tpu_sc as plsc

_N = 10000
_E = 320000
_D = 128

_NW = 32
_ROWS = 313
_NPAD = _NW * _ROWS
_BATCH = 128
_EPT = _E // _NW
_CAP = 10112
_BSTRIDE = _BATCH + 16

_mesh = plsc.VectorSubcoreMesh(core_axis_name="c", subcore_axis_name="s")


def _wid():
    return lax.axis_index("s") * 2 + lax.axis_index("c")


def _splat(x):
    return jnp.full((16,), 1, jnp.int32) * x


@functools.partial(
    pl.kernel,
    out_type=[
        jax.ShapeDtypeStruct((_NW * _NW * _CAP,), jnp.int32),
        jax.ShapeDtypeStruct((_NW * _NW * _CAP,), jnp.int32),
        jax.ShapeDtypeStruct((_NW * _NW * 16,), jnp.int32),
    ],
    mesh=_mesh,
    scratch_types=[
        pltpu.VMEM((_EPT + 16,), jnp.int32),
        pltpu.VMEM((_EPT + 16,), jnp.int32),
        pltpu.VMEM((_NW * _BSTRIDE,), jnp.int32),
        pltpu.VMEM((_NW * _BSTRIDE,), jnp.int32),
        pltpu.VMEM((_NW * 16,), jnp.int32),
        pltpu.VMEM((_NW * 16,), jnp.int32),
        pltpu.VMEM((16,), jnp.int32),
    ],
)
def _bucketize(edges_hbm, bsrc_hbm, bdst_hbm, cnt_hbm, sstage, dstage, sbuf,
               dbuf, fill, done, cbuf):
    t = _wid()
    zero16 = jnp.zeros((16,), jnp.int32)

    def zinit(b, _):
        fill[pl.ds(pl.multiple_of(b * 16, 16), 16)] = zero16
        done[pl.ds(pl.multiple_of(b * 16, 16), 16)] = zero16
        return 0

    lax.fori_loop(0, _NW, zinit, 0)

    eo = pl.multiple_of(t * _EPT, 16)
    pltpu.sync_copy(edges_hbm.at[pl.ds(eo, _EPT)], sstage.at[pl.ds(0, _EPT)])
    pltpu.sync_copy(edges_hbm.at[pl.ds(_E + eo, _EPT)],
                    dstage.at[pl.ds(0, _EPT)])

    def edge(e, _):
        s = sstage[pl.ds(e, 16)][0]
        d = dstage[pl.ds(e, 16)][0]
        b = d // _ROWS
        loc = d - b * _ROWS
        c = fill[pl.ds(b * 16, 16)][0]
        sbuf[pl.ds(b * _BSTRIDE + c, 16)] = _splat(s)
        dbuf[pl.ds(b * _BSTRIDE + c, 16)] = _splat(loc)

        def flush(_):
            w0 = done[pl.ds(b * 16, 16)][0]
            o = pl.multiple_of((b * _NW + t) * _CAP + w0, 8)
            bo = pl.multiple_of(b * _BSTRIDE, 8)
            pltpu.sync_copy(sbuf.at[pl.ds(bo, _BATCH)],
                            bsrc_hbm.at[pl.ds(o, _BATCH)])
            pltpu.sync_copy(dbuf.at[pl.ds(bo, _BATCH)],
                            bdst_hbm.at[pl.ds(o, _BATCH)])
            done[pl.ds(b * 16, 16)] = _splat(w0 + _BATCH)
            fill[pl.ds(b * 16, 16)] = zero16
            return 0

        def keep(_):
            fill[pl.ds(b * 16, 16)] = _splat(c + 1)
            return 0

        lax.cond(c + 1 >= _BATCH, flush, keep, 0)
        return 0

    lax.fori_loop(0, _EPT, edge, 0)

    pad_d = jnp.full((16,), _ROWS, jnp.int32)

    def fin(b, _):
        c = fill[pl.ds(b * 16, 16)][0]
        for j in range(8):
            @pl.when(c + 16 * j < _BATCH)
            def _():
                sbuf[pl.ds(b * _BSTRIDE + c + 16 * j, 16)] = zero16
                dbuf[pl.ds(b * _BSTRIDE + c + 16 * j, 16)] = pad_d
        w0 = done[pl.ds(b * 16, 16)][0]
        o = pl.multiple_of((b * _NW + t) * _CAP + w0, 8)
        bo = pl.multiple_of(b * _BSTRIDE, 8)
        pltpu.sync_copy(sbuf.at[pl.ds(bo, _BATCH)],
                        bsrc_hbm.at[pl.ds(o, _BATCH)])
        pltpu.sync_copy(dbuf.at[pl.ds(bo, _BATCH)],
                        bdst_hbm.at[pl.ds(o, _BATCH)])
        cbuf[pl.ds(0, 16)] = _splat(w0 + c)
        co = pl.multiple_of((b * _NW + t) * 16, 16)
        pltpu.sync_copy(cbuf, cnt_hbm.at[pl.ds(co, 16)])
        return 0

    lax.fori_loop(0, _NW, fin, 0)


@functools.partial(
    pl.kernel,
    out_type=[
        jax.ShapeDtypeStruct((_NPAD * _D,), jnp.float32),
        jax.ShapeDtypeStruct((_NPAD * 16,), jnp.float32),
    ],
    mesh=_mesh,
    scratch_types=[
        pltpu.VMEM(((_ROWS + 1) * _D,), jnp.float32),
        pltpu.VMEM(((_ROWS + 1) * 16,), jnp.float32),
        pltpu.VMEM((_BATCH,), jnp.int32),
        pltpu.VMEM((_BATCH + 16,), jnp.int32),
        pltpu.VMEM((_BATCH, _D), jnp.float32),
        pltpu.VMEM((_NW * 16,), jnp.int32),
        pltpu.SemaphoreType.DMA,
    ],
)
def _segsum(m_hbm, bsrc_hbm, bdst_hbm, cnt_hbm, out_hbm, deg_hbm, acc, dacc,
            sidx, didx, rows, cbuf, sem):
    w = _wid()
    zero = jnp.zeros((16,), jnp.float32)
    ones = jnp.ones((16,), jnp.float32)

    def zbody(r, _):
        for j in range(_D // 16):
            acc[pl.ds(r * _D + 16 * j, 16)] = zero
        dacc[pl.ds(r * 16, 16)] = zero
        return 0

    lax.fori_loop(0, _ROWS + 1, zbody, 0)

    pltpu.sync_copy(
        cnt_hbm.at[pl.ds(pl.multiple_of(w * _NW * 16, 16), _NW * 16)], cbuf)

    def seg(tt, _):
        cnt = cbuf[pl.ds(tt * 16, 16)][0]
        nb = (cnt + (_BATCH - 1)) // _BATCH
        segbase = (w * _NW + tt) * _CAP

        def batch(b, _):
            o = pl.multiple_of(segbase + b * _BATCH, 8)
            pltpu.sync_copy(bsrc_hbm.at[pl.ds(o, _BATCH)], sidx)
            pltpu.sync_copy(bdst_hbm.at[pl.ds(o, _BATCH)],
                            didx.at[pl.ds(0, _BATCH)])
            pltpu.async_copy(m_hbm.at[sidx], rows, sem).wait()

            def group(g, _):
                dvec = didx[pl.ds(g * 16, 16)]
                bases = [dvec[i] * _D for i in range(16)]
                for i in range(16):
                    plsc.addupdate(dacc.at[pl.ds(dvec[i] * 16, 16)], ones)
                for j in range(_D // 16):
                    for i in range(16):
                        plsc.addupdate(
                            acc.at[pl.ds(bases[i] + 16 * j, 16)],
                            rows[g * 16 + i, pl.ds(16 * j, 16)])
                return 0

            lax.fori_loop(0, _BATCH // 16, group, 0)
            return 0

        lax.fori_loop(0, nb, batch, 0)
        return 0

    lax.fori_loop(0, _NW, seg, 0)

    pltpu.sync_copy(acc.at[pl.ds(0, _ROWS * _D)],
                    out_hbm.at[pl.ds(pl.multiple_of(w * _ROWS * _D, 64),
                                     _ROWS * _D)])
    pltpu.sync_copy(dacc.at[pl.ds(0, _ROWS * 16)],
                    deg_hbm.at[pl.ds(pl.multiple_of(w * _ROWS * 16, 16),
                                     _ROWS * 16)])


_R = 1000


def _gelu(x):
    return 0.5 * x * (1.0 + lax.erf(x * 0.7071067811865476))


def _mm(a, b):
    return jnp.dot(a, b, preferred_element_type=jnp.float32)


def _head_body(f_ref, Win_ref, bin_ref, W1_ref, b1_ref, h_ref, m_ref):
    h = _gelu(_mm(f_ref[...], Win_ref[...]) + bin_ref[...])
    h_ref[...] = h
    m_ref[...] = _gelu(_mm(h, W1_ref[...]) + b1_ref[...])


def _mid_body(s_ref, dg_ref, h_ref, W2_ref, b2_ref, W3_ref, b3_ref, Wl_ref,
              bl_ref, W1n_ref, b1n_ref, hn_ref, mn_ref):
    h = h_ref[...]
    inv = 1.0 / jnp.maximum(dg_ref[...][:, :1], 1.0)
    agg = s_ref[...] * inv
    t = _gelu(_mm(agg, W2_ref[...]) + b2_ref[...] + _mm(h, W3_ref[...]) +
              b3_ref[...])
    hn = _mm(t, Wl_ref[...]) + bl_ref[...] + h
    hn_ref[...] = hn
    mn_ref[...] = _gelu(_mm(hn, W1n_ref[...]) + b1n_ref[...])


def _tail_body(s_ref, dg_ref, h_ref, W2_ref, b2_ref, W3_ref, b3_ref, Wl_ref,
               bl_ref, Wo_ref, bo_ref, o_ref):
    h = h_ref[...]
    inv = 1.0 / jnp.maximum(dg_ref[...][:, :1], 1.0)
    agg = s_ref[...] * inv
    t = _gelu(_mm(agg, W2_ref[...]) + b2_ref[...] + _mm(h, W3_ref[...]) +
              b3_ref[...])
    hn = _mm(t, Wl_ref[...]) + bl_ref[...] + h
    o_ref[...] = _mm(hn, Wo_ref[...]) + bo_ref[...]


_rows_spec = pl.BlockSpec((_R, _D), lambda i: (i, 0))
_deg_spec = pl.BlockSpec((_R, 16), lambda i: (i, 0))
_w_spec = pl.BlockSpec((_D, _D), lambda i: (0, 0))
_b_spec = pl.BlockSpec((_D,), lambda i: (0,))
_row_out = jax.ShapeDtypeStruct((_N, _D), jnp.float32)

_head = pl.pallas_call(
    _head_body,
    grid=(_N // _R,),
    in_specs=[_rows_spec, _w_spec, _b_spec, _w_spec, _b_spec],
    out_specs=[_rows_spec, _rows_spec],
    out_shape=[_row_out, _row_out],
)

_mid = pl.pallas_call(
    _mid_body,
    grid=(_N // _R,),
    in_specs=[_rows_spec, _deg_spec, _rows_spec] + [_w_spec, _b_spec] * 4,
    out_specs=[_rows_spec, _rows_spec],
    out_shape=[_row_out, _row_out],
)

_tail = pl.pallas_call(
    _tail_body,
    grid=(_N // _R,),
    in_specs=[_rows_spec, _deg_spec, _rows_spec] + [_w_spec, _b_spec] * 4,
    out_specs=_rows_spec,
    out_shape=_row_out,
)


def kernel(feats, edge_index, W_in, b_in,
           W1_0, b1_0, W2_0, b2_0, W3_0, b3_0, Wl_0, bl_0,
           W1_1, b1_1, W2_1, b2_1, W3_1, b3_1, Wl_1, bl_1,
           W_out, b_out):
    bsrc, bdst, cnts = _bucketize(edge_index.reshape(-1))
    h, m0 = _head(feats, W_in, b_in, W1_0, b1_0)
    s0, dg = _segsum(m0, bsrc, bdst, cnts)
    dg = dg.reshape(_NPAD, 16)[:_N]
    h1, m1 = _mid(s0.reshape(_NPAD, _D)[:_N], dg, h, W2_0, b2_0, W3_0, b3_0,
                  Wl_0, bl_0, W1_1, b1_1)
    s1, _ = _segsum(m1, bsrc, bdst, cnts)
    out = _tail(s1.reshape(_NPAD, _D)[:_N], dg, h1, W2_1, b2_1, W3_1, b3_1,
                Wl_1, bl_1, W_out, b_out)
    return out
